# baseline (speedup 1.0000x reference)
import jax
import jax.numpy as jnp
from jax import lax
from jax.experimental import pallas as pl
from jax.experimental.pallas import tpu as pltpu
from jax.experimental.pallas import tpu_sc as plsc

_B, _C, _H, _W = 32, 2, 512, 512
_K = 500
_KPAD = 512
_NIDX = _C * _KPAD
_LANES = 16
_BLK = 8
_MOFF = _KPAD
_TOFF = 2 * _KPAD
_BLOB = 4 * _KPAD


def _tec_body(feat_ref, blob_ref, out_ref, stage_ref,
              blob_v, vals8_v, pos_v, vals_v, part_v, sem, sem2):
    c = lax.axis_index("c")
    s = lax.axis_index("s")
    b = s * 2 + c

    pltpu.sync_copy(blob_ref.at[b], blob_v)

    row0_base = b * _C * _H

    def issue(i, carry):
        iv = blob_v[pl.ds(pl.multiple_of(i * _LANES, _LANES), _LANES)]
        for j in range(_LANES):
            p = iv[j]
            k = i * _LANES + j
            h = lax.shift_right_logical(p, 9)
            w8 = pl.multiple_of(p & (_W - 1) & ~(_BLK - 1), _BLK)
            dst0 = pl.multiple_of(k * _BLK, _BLK)
            dst1 = pl.multiple_of((_KPAD + k) * _BLK, _BLK)
            r0 = row0_base + h
            pltpu.async_copy(feat_ref.at[r0, pl.ds(w8, _BLK)],
                             vals8_v.at[pl.ds(dst0, _BLK)], sem)
            pltpu.async_copy(feat_ref.at[r0 + _H, pl.ds(w8, _BLK)],
                             vals8_v.at[pl.ds(dst1, _BLK)], sem)
        return carry

    lax.fori_loop(0, _KPAD // _LANES, issue, 0)

    lane_ids = lax.iota(jnp.int32, _LANES)
    for i in range(_KPAD // _LANES):
        sl = pl.ds(i * _LANES, _LANES)
        l = blob_v[sl] & (_BLK - 1)
        k_vec = lane_ids + i * _LANES + b * _NIDX
        pos_v[sl] = k_vec * _BLK + l
        pos_v[pl.ds(_KPAD + i * _LANES, _LANES)] = (
            (k_vec + _KPAD) * _BLK + l)

    for j in range(_NIDX * _BLK // _W):
        pltpu.make_async_copy(feat_ref.at[0],
                              vals8_v.at[pl.ds(j * _W, _W)], sem).wait()

    pltpu.sync_copy(vals8_v,
                    stage_ref.at[pl.ds(b * _NIDX * _BLK, _NIDX * _BLK)])
    copies = []
    for j in range(_NIDX // 128):
        sl = pl.ds(j * 128, 128)
        copies.append(pltpu.async_copy(stage_ref.at[pos_v.at[sl]],
                                       vals_v.at[sl], sem2))
    for cp in copies:
        cp.wait()

    acc = jnp.zeros((_LANES,), jnp.float32)
    macc = jnp.zeros((_LANES,), jnp.float32)
    for i in range(_KPAD // _LANES):
        sl0 = pl.ds(i * _LANES, _LANES)
        sl1 = pl.ds(_KPAD + i * _LANES, _LANES)
        m = blob_v[pl.ds(_MOFF + i * _LANES, _LANES)].astype(jnp.float32)
        t0 = plsc.bitcast(blob_v[pl.ds(_TOFF + i * _LANES, _LANES)],
                          jnp.float32)
        t1 = plsc.bitcast(blob_v[pl.ds(_TOFF + _KPAD + i * _LANES, _LANES)],
                          jnp.float32)
        d0 = jnp.abs(vals_v[sl0] - t0)
        d1 = jnp.abs(vals_v[sl1] - t1)
        acc = acc + (d0 + d1) * m
        macc = macc + m

    part_v[pl.ds(0, _LANES)] = acc
    part_v[pl.ds(_LANES, _LANES)] = macc
    pltpu.sync_copy(part_v, out_ref.at[b])


@jax.jit
def kernel(output, mask, ind, target):
    feat = output.reshape(_B * _C * _H, _W)
    tgt_bits = lax.bitcast_convert_type(
        jnp.transpose(target, (0, 2, 1)), jnp.int32)
    blob = jnp.zeros((_B, _BLOB), jnp.int32)
    blob = blob.at[:, :_K].set(ind)
    blob = blob.at[:, _MOFF:_MOFF + _K].set(mask.astype(jnp.int32))
    blob = blob.at[:, _TOFF:_TOFF + _K].set(tgt_bits[:, 0])
    blob = blob.at[:, _TOFF + _KPAD:_TOFF + _KPAD + _K].set(tgt_bits[:, 1])

    mesh = plsc.VectorSubcoreMesh(core_axis_name="c", subcore_axis_name="s")
    f = pl.kernel(
        _tec_body,
        mesh=mesh,
        compiler_params=pltpu.CompilerParams(needs_layout_passes=False),
        out_type=(
            jax.ShapeDtypeStruct((_B, 2 * _LANES), jnp.float32),
            jax.ShapeDtypeStruct((_B * _NIDX * _BLK,), jnp.float32),
        ),
        scratch_types=[
            pltpu.VMEM((_BLOB,), jnp.int32),
            pltpu.VMEM((_NIDX * _BLK,), jnp.float32),
            pltpu.VMEM((_NIDX,), jnp.int32),
            pltpu.VMEM((_NIDX,), jnp.float32),
            pltpu.VMEM((2 * _LANES,), jnp.float32),
            pltpu.SemaphoreType.DMA,
            pltpu.SemaphoreType.DMA,
        ],
    )
    parts, _ = f(feat, blob)
    loss = jnp.sum(parts[:, :_LANES]) / (
        _C * jnp.sum(parts[:, _LANES:]) + 1e-4)
    return loss


# speedup vs baseline: 1.2213x; 1.2213x over previous
import jax
import jax.numpy as jnp
from jax import lax
from jax.experimental import pallas as pl
from jax.experimental.pallas import tpu as pltpu
from jax.experimental.pallas import tpu_sc as plsc

_B, _C, _H, _W = 32, 2, 512, 512
_K = 500
_KPAD = 512
_NIDX = _C * _KPAD
_LANES = 16
_BLK = 8
_MOFF = _KPAD
_TOFF = 2 * _KPAD
_BLOB = 4 * _KPAD


def _tec_body(feat_ref, blob_ref, out_ref, stage_ref,
              blob_v, vals8_v, pos_v, vals_v, part_v, sem, sem2):
    c = lax.axis_index("c")
    s = lax.axis_index("s")
    b = s * 2 + c

    pltpu.sync_copy(blob_ref.at[b], blob_v)

    row0_base = b * _C * _H

    def issue(i, carry):
        iv = blob_v[pl.ds(pl.multiple_of(i * _LANES, _LANES), _LANES)]
        for j in range(_LANES):
            p = iv[j]
            k = i * _LANES + j
            h = lax.shift_right_logical(p, 9)
            w8 = pl.multiple_of(p & (_W - 1) & ~(_BLK - 1), _BLK)
            dst0 = pl.multiple_of(k * _BLK, _BLK)
            dst1 = pl.multiple_of((_KPAD + k) * _BLK, _BLK)
            r0 = row0_base + h
            pltpu.async_copy(feat_ref.at[r0, pl.ds(w8, _BLK)],
                             vals8_v.at[pl.ds(dst0, _BLK)], sem)
            pltpu.async_copy(feat_ref.at[r0 + _H, pl.ds(w8, _BLK)],
                             vals8_v.at[pl.ds(dst1, _BLK)], sem)
        return carry

    lax.fori_loop(0, _KPAD // _LANES, issue, 0)

    lane_ids = lax.iota(jnp.int32, _LANES)
    for i in range(_KPAD // _LANES):
        sl = pl.ds(i * _LANES, _LANES)
        l = blob_v[sl] & (_BLK - 1)
        k_vec = lane_ids + i * _LANES + b * _NIDX
        pos_v[sl] = k_vec * _BLK + l
        pos_v[pl.ds(_KPAD + i * _LANES, _LANES)] = (
            (k_vec + _KPAD) * _BLK + l)

    for j in range(_NIDX * _BLK // _W):
        pltpu.make_async_copy(feat_ref.at[0],
                              vals8_v.at[pl.ds(j * _W, _W)], sem).wait()

    pltpu.sync_copy(vals8_v,
                    stage_ref.at[pl.ds(b * _NIDX * _BLK, _NIDX * _BLK)])
    copies = []
    for j in range(_NIDX // 128):
        sl = pl.ds(j * 128, 128)
        copies.append(pltpu.async_copy(stage_ref.at[pos_v.at[sl]],
                                       vals_v.at[sl], sem2))
    for cp in copies:
        cp.wait()

    acc = jnp.zeros((_LANES,), jnp.float32)
    macc = jnp.zeros((_LANES,), jnp.float32)
    for i in range(_KPAD // _LANES):
        sl0 = pl.ds(i * _LANES, _LANES)
        sl1 = pl.ds(_KPAD + i * _LANES, _LANES)
        m = blob_v[pl.ds(_MOFF + i * _LANES, _LANES)].astype(jnp.float32)
        t0 = plsc.bitcast(blob_v[pl.ds(_TOFF + i * _LANES, _LANES)],
                          jnp.float32)
        t1 = plsc.bitcast(blob_v[pl.ds(_TOFF + _KPAD + i * _LANES, _LANES)],
                          jnp.float32)
        d0 = jnp.abs(vals_v[sl0] - t0)
        d1 = jnp.abs(vals_v[sl1] - t1)
        acc = acc + (d0 + d1) * m
        macc = macc + m

    part_v[pl.ds(0, _LANES)] = acc
    part_v[pl.ds(_LANES, _LANES)] = macc
    pltpu.sync_copy(part_v, out_ref.at[b])


@jax.jit
def kernel(output, mask, ind, target):
    feat = output.reshape(_B * _C * _H, _W)
    tgt_bits = lax.bitcast_convert_type(
        jnp.transpose(target, (0, 2, 1)), jnp.int32)
    pad = ((0, 0), (0, _KPAD - _K))
    blob = jnp.concatenate([
        jnp.pad(ind, pad),
        jnp.pad(mask.astype(jnp.int32), pad),
        jnp.pad(tgt_bits[:, 0], pad),
        jnp.pad(tgt_bits[:, 1], pad),
    ], axis=1)

    mesh = plsc.VectorSubcoreMesh(core_axis_name="c", subcore_axis_name="s")
    f = pl.kernel(
        _tec_body,
        mesh=mesh,
        compiler_params=pltpu.CompilerParams(needs_layout_passes=False),
        out_type=(
            jax.ShapeDtypeStruct((_B, 2 * _LANES), jnp.float32),
            jax.ShapeDtypeStruct((_B * _NIDX * _BLK,), jnp.float32),
        ),
        scratch_types=[
            pltpu.VMEM((_BLOB,), jnp.int32),
            pltpu.VMEM((_NIDX * _BLK,), jnp.float32),
            pltpu.VMEM((_NIDX,), jnp.int32),
            pltpu.VMEM((_NIDX,), jnp.float32),
            pltpu.VMEM((2 * _LANES,), jnp.float32),
            pltpu.SemaphoreType.DMA,
            pltpu.SemaphoreType.DMA,
        ],
    )
    parts, _ = f(feat, blob)
    loss = jnp.sum(parts[:, :_LANES]) / (
        _C * jnp.sum(parts[:, _LANES:]) + 1e-4)
    return loss


# speedup vs baseline: 1.3104x; 1.0729x over previous
import jax
import jax.numpy as jnp
from jax import lax
from jax.experimental import pallas as pl
from jax.experimental.pallas import tpu as pltpu
from jax.experimental.pallas import tpu_sc as plsc

_B, _C, _H, _W = 32, 2, 512, 512
_K = 500
_KPAD = 512
_NIDX = _C * _KPAD
_LANES = 16
_BLK = 8
_MOFF = _KPAD
_TOFF = 2 * _KPAD
_BLOB = 4 * _KPAD


def _tec_body(feat_ref, blob_ref, out_ref,
              blob_v, vals8_v, part_v, sem):
    c = lax.axis_index("c")
    s = lax.axis_index("s")
    b = s * 2 + c

    pltpu.sync_copy(blob_ref.at[b], blob_v)

    row0_base = b * _C * _H

    def issue(i, carry):
        iv = blob_v[pl.ds(pl.multiple_of(i * _LANES, _LANES), _LANES)]
        for j in range(_LANES):
            p = iv[j]
            k = i * _LANES + j
            h = lax.shift_right_logical(p, 9)
            w8 = pl.multiple_of(p & (_W - 1) & ~(_BLK - 1), _BLK)
            dst0 = pl.multiple_of(k * _BLK, _BLK)
            dst1 = pl.multiple_of((_KPAD + k) * _BLK, _BLK)
            r0 = row0_base + h
            pltpu.async_copy(feat_ref.at[r0, pl.ds(w8, _BLK)],
                             vals8_v.at[pl.ds(dst0, _BLK)], sem)
            pltpu.async_copy(feat_ref.at[r0 + _H, pl.ds(w8, _BLK)],
                             vals8_v.at[pl.ds(dst1, _BLK)], sem)
        return carry

    lax.fori_loop(0, _KPAD // _LANES, issue, 0)

    for j in range(_NIDX * _BLK // _W):
        pltpu.make_async_copy(feat_ref.at[0],
                              vals8_v.at[pl.ds(j * _W, _W)], sem).wait()

    lane_ids = lax.iota(jnp.int32, _LANES)
    acc = jnp.zeros((_LANES,), jnp.float32)
    macc = jnp.zeros((_LANES,), jnp.float32)
    for i in range(_KPAD // _LANES):
        l = blob_v[pl.ds(i * _LANES, _LANES)] & (_BLK - 1)
        k_vec = lane_ids + i * _LANES
        p0 = plsc.load_gather(vals8_v, [k_vec * _BLK + l])
        p1 = plsc.load_gather(vals8_v, [(k_vec + _KPAD) * _BLK + l])
        m = blob_v[pl.ds(_MOFF + i * _LANES, _LANES)].astype(jnp.float32)
        t0 = plsc.bitcast(blob_v[pl.ds(_TOFF + i * _LANES, _LANES)],
                          jnp.float32)
        t1 = plsc.bitcast(blob_v[pl.ds(_TOFF + _KPAD + i * _LANES, _LANES)],
                          jnp.float32)
        d0 = jnp.abs(p0 - t0)
        d1 = jnp.abs(p1 - t1)
        acc = acc + (d0 + d1) * m
        macc = macc + m

    part_v[pl.ds(0, _LANES)] = acc
    part_v[pl.ds(_LANES, _LANES)] = macc
    pltpu.sync_copy(part_v, out_ref.at[b])


@jax.jit
def kernel(output, mask, ind, target):
    feat = output.reshape(_B * _C * _H, _W)
    tgt_bits = lax.bitcast_convert_type(
        jnp.transpose(target, (0, 2, 1)), jnp.int32)
    pad = ((0, 0), (0, _KPAD - _K))
    blob = jnp.concatenate([
        jnp.pad(ind, pad),
        jnp.pad(mask.astype(jnp.int32), pad),
        jnp.pad(tgt_bits[:, 0], pad),
        jnp.pad(tgt_bits[:, 1], pad),
    ], axis=1)

    mesh = plsc.VectorSubcoreMesh(core_axis_name="c", subcore_axis_name="s")
    f = pl.kernel(
        _tec_body,
        mesh=mesh,
        compiler_params=pltpu.CompilerParams(needs_layout_passes=False),
        out_type=jax.ShapeDtypeStruct((_B, 2 * _LANES), jnp.float32),
        scratch_types=[
            pltpu.VMEM((_BLOB,), jnp.int32),
            pltpu.VMEM((_NIDX * _BLK,), jnp.float32),
            pltpu.VMEM((2 * _LANES,), jnp.float32),
            pltpu.SemaphoreType.DMA,
        ],
    )
    parts = f(feat, blob)
    loss = jnp.sum(parts[:, :_LANES]) / (
        _C * jnp.sum(parts[:, _LANES:]) + 1e-4)
    return loss
